# baseline (speedup 1.0000x reference)
import functools

import jax
import jax.numpy as jnp
from jax import lax
from jax.experimental import pallas as pl
from jax.experimental.pallas import tpu as pltpu
from jax.experimental.pallas import tpu_sc as plsc

EMBED_DIM = 64
OUT_W = 2 * EMBED_DIM
CHUNK = 128
K = 2
NBUF = 4
TLANES = 384
NLANE = 16


@functools.lru_cache(maxsize=None)
def _build_transpose(vocab):
    info = plsc.get_sparse_core_info()
    nc, ns = info.num_cores, info.num_subcores
    nw = nc * ns
    main = (vocab // CHUNK) * CHUNK
    tail = vocab - main
    n_chunks = main // TLANES
    assert main % TLANES == 0
    n_rounds = -(-n_chunks // nw)

    mesh = plsc.VectorSubcoreMesh(core_axis_name="c", subcore_axis_name="s")

    @functools.partial(
        pl.kernel,
        out_type=jax.ShapeDtypeStruct((vocab // 2, OUT_W), jnp.float32),
        mesh=mesh,
        scratch_types=[
            [pltpu.VMEM((EMBED_DIM, TLANES), jnp.float32) for _ in range(2)],
            [pltpu.VMEM((TLANES // 2, OUT_W), jnp.float32) for _ in range(2)],
            pltpu.VMEM((tail * EMBED_DIM // OUT_W, OUT_W), jnp.float32)
            if tail else None,
            [pltpu.SemaphoreType.DMA for _ in range(2)],
            [pltpu.SemaphoreType.DMA for _ in range(2)],
        ],
        compiler_params=pltpu.CompilerParams(needs_layout_passes=False),
    )
    def transpose_kernel(embT_hbm, tail_hbm, out_hbm, in_v, out_v, tail_v,
                         isem, osem):
        wid = lax.axis_index("s") * nc + lax.axis_index("c")

        ramp = lax.iota(jnp.int32, NLANE)
        row_pat = lax.shift_right_logical(ramp, 1)
        par_pat = lax.mul(lax.rem(ramp, 2), jnp.int32(EMBED_DIM))

        def chunk_of(t):
            return wid + t * nw

        def fire_in(c, b):
            pltpu.async_copy(embT_hbm.at[:, pl.ds(c * TLANES, TLANES)],
                             in_v[b], isem[b])

        def wait_in(c, b):
            pltpu.make_async_copy(embT_hbm.at[:, pl.ds(c * TLANES, TLANES)],
                                  in_v[b], isem[b]).wait()

        def start_out(c, b):
            pltpu.async_copy(out_v[b],
                             out_hbm.at[pl.ds(c * (TLANES // 2), TLANES // 2)],
                             osem[b])

        def wait_out(c, b):
            pltpu.make_async_copy(out_v[b],
                                  out_hbm.at[pl.ds(c * (TLANES // 2),
                                                   TLANES // 2)],
                                  osem[b]).wait()

        def compute(b):
            @pl.loop(0, EMBED_DIM)
            def _(j):
                f_vec = lax.rem(ramp + j, jnp.int32(EMBED_DIM))
                cols = par_pat + f_vec
                for v0 in range(0, TLANES, NLANE):
                    vals = plsc.load_gather(in_v[b], [f_vec, ramp + v0])
                    plsc.store_scatter(
                        out_v[b], [row_pat + (v0 // 2), cols], vals)

        @pl.when(chunk_of(0) < n_chunks)
        def _():
            fire_in(chunk_of(0), 0)

        @pl.loop(0, n_rounds)
        def _(t):
            b = lax.rem(t, 2)
            c = chunk_of(t)

            @pl.when(c < n_chunks)
            def _():
                for bb in range(2):
                    @pl.when(b == bb)
                    def _():
                        wait_in(c, bb)

                        @pl.when(t >= 2)
                        def _():
                            wait_out(chunk_of(t - 2), bb)

                        @pl.when(chunk_of(t + 1) < n_chunks)
                        def _():
                            fire_in(chunk_of(t + 1), 1 - bb)

                        compute(bb)
                        start_out(c, bb)

        my_n = lax.div(n_chunks - wid + nw - 1, nw)

        @pl.loop(0, 2)
        def _(i):
            t_last = my_n - 2 + i

            @pl.when(t_last >= 0)
            def _():
                for bb in range(2):
                    @pl.when(lax.rem(t_last, 2) == bb)
                    def _():
                        wait_out(chunk_of(t_last), bb)

        if tail:
            @pl.when(wid == 0)
            def _():
                pltpu.sync_copy(tail_hbm, tail_v)
                pltpu.sync_copy(
                    tail_v,
                    out_hbm.at[pl.ds(main // 2, tail * EMBED_DIM // OUT_W)])

    return transpose_kernel, tail


@functools.lru_cache(maxsize=None)
def _build_gather(n_total):
    info = plsc.get_sparse_core_info()
    nc, ns = info.num_cores, info.num_subcores
    nw = nc * ns
    per_w = n_total // nw
    group = K * CHUNK
    assert per_w * nw == n_total and per_w % group == 0
    n_chunks = per_w // CHUNK
    n_groups = per_w // group
    assert n_groups % NBUF == 0

    mesh = plsc.VectorSubcoreMesh(core_axis_name="c", subcore_axis_name="s")

    @functools.partial(
        pl.kernel,
        out_type=jax.ShapeDtypeStruct((n_total, OUT_W), jnp.float32),
        mesh=mesh,
        scratch_types=[
            pltpu.VMEM((n_chunks, CHUNK), jnp.int32),
            [pltpu.VMEM((group, EMBED_DIM), jnp.float32) for _ in range(NBUF)],
            [pltpu.SemaphoreType.DMA for _ in range(NBUF)],
            [pltpu.SemaphoreType.DMA for _ in range(NBUF)],
        ],
        compiler_params=pltpu.CompilerParams(use_tc_tiling_on_sc=False),
    )
    def gather_kernel(idx_hbm, table_hbm, out_hbm, idx_v, rows, gsem, wsem):
        wid = lax.axis_index("s") * nc + lax.axis_index("c")
        base = wid * per_w

        pltpu.sync_copy(idx_hbm.at[wid], idx_v)

        def out_slice(g):
            return out_hbm.at[pl.ds(base + g * group, group),
                              pl.ds(0, EMBED_DIM)]

        def fire(g, b):
            for t in range(K):
                pltpu.async_copy(
                    table_hbm.at[idx_v.at[g * K + t]],
                    rows[b].at[pl.ds(t * CHUNK, CHUNK)],
                    gsem[b])

        def drain_gathers(g, b):
            for t in range(K):
                pltpu.make_async_copy(
                    table_hbm.at[idx_v.at[g * K + t]],
                    rows[b].at[pl.ds(t * CHUNK, CHUNK)],
                    gsem[b]).wait()

        def start_write(g, b):
            pltpu.async_copy(rows[b], out_slice(g), wsem[b])

        def wait_write(g, b):
            pltpu.make_async_copy(rows[b], out_slice(g), wsem[b]).wait()

        fire(0, 0)

        @pl.loop(0, n_groups, step=NBUF)
        def _(g0):
            for b in range(NBUF):
                g = g0 + b
                drain_gathers(g, b)
                nb = (b + 1) % NBUF

                @pl.when(g + 1 < n_groups)
                def _():
                    @pl.when(g + 1 >= NBUF)
                    def _():
                        wait_write(g + 1 - NBUF, nb)
                    fire(g + 1, nb)

                start_write(g, b)

        for b in range(NBUF):
            wait_write(n_groups - NBUF + b, b)

    return gather_kernel, nw, n_chunks


def kernel(indices, embeddings):
    batch, hist = indices.shape
    vocab = embeddings.shape[0]
    n_total = batch * hist

    transpose, tail = _build_transpose(vocab)
    run, nw, n_chunks = _build_gather(n_total)

    main = vocab - tail
    t_rm = transpose(embeddings.T,
                     embeddings[main:].reshape(tail * EMBED_DIM // OUT_W,
                                               OUT_W))
    t_lin = t_rm.reshape(vocab, EMBED_DIM)

    idx3 = indices.reshape(nw, n_chunks, CHUNK)
    out2 = run(idx3, t_lin)
    return out2[:, :EMBED_DIM].reshape(batch, hist, EMBED_DIM)


# speedup vs baseline: 1.0292x; 1.0292x over previous
import functools

import jax
import jax.numpy as jnp
from jax import lax
from jax.experimental import pallas as pl
from jax.experimental.pallas import tpu as pltpu
from jax.experimental.pallas import tpu_sc as plsc

EMBED_DIM = 64
OUT_W = 2 * EMBED_DIM
CHUNK = 128
K = 4
NBUF = 2
TLANES = 384
NLANE = 16


@functools.lru_cache(maxsize=None)
def _build_transpose(vocab):
    info = plsc.get_sparse_core_info()
    nc, ns = info.num_cores, info.num_subcores
    nw = nc * ns
    main = (vocab // CHUNK) * CHUNK
    tail = vocab - main
    n_chunks = main // TLANES
    assert main % TLANES == 0
    n_rounds = -(-n_chunks // nw)

    mesh = plsc.VectorSubcoreMesh(core_axis_name="c", subcore_axis_name="s")

    @functools.partial(
        pl.kernel,
        out_type=jax.ShapeDtypeStruct((vocab // 2, OUT_W), jnp.float32),
        mesh=mesh,
        scratch_types=[
            [pltpu.VMEM((EMBED_DIM, TLANES), jnp.float32) for _ in range(2)],
            [pltpu.VMEM((TLANES // 2, OUT_W), jnp.float32) for _ in range(2)],
            pltpu.VMEM((tail * EMBED_DIM // OUT_W, OUT_W), jnp.float32)
            if tail else None,
            [pltpu.SemaphoreType.DMA for _ in range(2)],
            [pltpu.SemaphoreType.DMA for _ in range(2)],
        ],
        compiler_params=pltpu.CompilerParams(needs_layout_passes=False),
    )
    def transpose_kernel(embT_hbm, tail_hbm, out_hbm, in_v, out_v, tail_v,
                         isem, osem):
        wid = lax.axis_index("s") * nc + lax.axis_index("c")

        ramp = lax.iota(jnp.int32, NLANE)
        row_pat = lax.shift_right_logical(ramp, 1)
        par_pat = lax.mul(lax.rem(ramp, 2), jnp.int32(EMBED_DIM))

        def chunk_of(t):
            return wid + t * nw

        def fire_in(c, b):
            pltpu.async_copy(embT_hbm.at[:, pl.ds(c * TLANES, TLANES)],
                             in_v[b], isem[b])

        def wait_in(c, b):
            pltpu.make_async_copy(embT_hbm.at[:, pl.ds(c * TLANES, TLANES)],
                                  in_v[b], isem[b]).wait()

        def start_out(c, b):
            pltpu.async_copy(out_v[b],
                             out_hbm.at[pl.ds(c * (TLANES // 2), TLANES // 2)],
                             osem[b])

        def wait_out(c, b):
            pltpu.make_async_copy(out_v[b],
                                  out_hbm.at[pl.ds(c * (TLANES // 2),
                                                   TLANES // 2)],
                                  osem[b]).wait()

        def compute(b):
            @pl.loop(0, EMBED_DIM, step=2)
            def _(j):
                for dj in range(2):
                    f_vec = lax.rem(ramp + (j + dj), jnp.int32(EMBED_DIM))
                    cols = par_pat + f_vec
                    for v0 in range(0, TLANES, NLANE):
                        vals = plsc.load_gather(in_v[b], [f_vec, ramp + v0])
                        plsc.store_scatter(
                            out_v[b], [row_pat + (v0 // 2), cols], vals)

        @pl.when(chunk_of(0) < n_chunks)
        def _():
            fire_in(chunk_of(0), 0)

        @pl.loop(0, n_rounds)
        def _(t):
            b = lax.rem(t, 2)
            c = chunk_of(t)

            @pl.when(c < n_chunks)
            def _():
                for bb in range(2):
                    @pl.when(b == bb)
                    def _():
                        wait_in(c, bb)

                        @pl.when(t >= 2)
                        def _():
                            wait_out(chunk_of(t - 2), bb)

                        @pl.when(chunk_of(t + 1) < n_chunks)
                        def _():
                            fire_in(chunk_of(t + 1), 1 - bb)

                        compute(bb)
                        start_out(c, bb)

        my_n = lax.div(n_chunks - wid + nw - 1, nw)

        @pl.loop(0, 2)
        def _(i):
            t_last = my_n - 2 + i

            @pl.when(t_last >= 0)
            def _():
                for bb in range(2):
                    @pl.when(lax.rem(t_last, 2) == bb)
                    def _():
                        wait_out(chunk_of(t_last), bb)

        if tail:
            @pl.when(wid == 0)
            def _():
                pltpu.sync_copy(tail_hbm, tail_v)
                pltpu.sync_copy(
                    tail_v,
                    out_hbm.at[pl.ds(main // 2, tail * EMBED_DIM // OUT_W)])

    return transpose_kernel, tail


@functools.lru_cache(maxsize=None)
def _build_gather(n_total):
    info = plsc.get_sparse_core_info()
    nc, ns = info.num_cores, info.num_subcores
    nw = nc * ns
    per_w = n_total // nw
    group = K * CHUNK
    assert per_w * nw == n_total and per_w % group == 0
    n_chunks = per_w // CHUNK
    n_groups = per_w // group
    assert n_groups % NBUF == 0

    mesh = plsc.VectorSubcoreMesh(core_axis_name="c", subcore_axis_name="s")

    @functools.partial(
        pl.kernel,
        out_type=jax.ShapeDtypeStruct((n_total, OUT_W), jnp.float32),
        mesh=mesh,
        scratch_types=[
            pltpu.VMEM((n_chunks, CHUNK), jnp.int32),
            [pltpu.VMEM((group, EMBED_DIM), jnp.float32) for _ in range(NBUF)],
            [pltpu.SemaphoreType.DMA for _ in range(NBUF)],
            [pltpu.SemaphoreType.DMA for _ in range(NBUF)],
        ],
        compiler_params=pltpu.CompilerParams(use_tc_tiling_on_sc=False),
    )
    def gather_kernel(idx_hbm, table_hbm, out_hbm, idx_v, rows, gsem, wsem):
        wid = lax.axis_index("s") * nc + lax.axis_index("c")
        base = wid * per_w

        pltpu.sync_copy(idx_hbm.at[wid], idx_v)

        def out_slice(g):
            return out_hbm.at[pl.ds(base + g * group, group),
                              pl.ds(0, EMBED_DIM)]

        def fire(g, b):
            for t in range(K):
                pltpu.async_copy(
                    table_hbm.at[idx_v.at[g * K + t]],
                    rows[b].at[pl.ds(t * CHUNK, CHUNK)],
                    gsem[b])

        def drain_gathers(g, b):
            for t in range(K):
                pltpu.make_async_copy(
                    table_hbm.at[idx_v.at[g * K + t]],
                    rows[b].at[pl.ds(t * CHUNK, CHUNK)],
                    gsem[b]).wait()

        def start_write(g, b):
            pltpu.async_copy(rows[b], out_slice(g), wsem[b])

        def wait_write(g, b):
            pltpu.make_async_copy(rows[b], out_slice(g), wsem[b]).wait()

        fire(0, 0)

        @pl.loop(0, n_groups, step=NBUF)
        def _(g0):
            for b in range(NBUF):
                g = g0 + b
                drain_gathers(g, b)
                nb = (b + 1) % NBUF

                @pl.when(g + 1 < n_groups)
                def _():
                    @pl.when(g + 1 >= NBUF)
                    def _():
                        wait_write(g + 1 - NBUF, nb)
                    fire(g + 1, nb)

                start_write(g, b)

        for b in range(NBUF):
            wait_write(n_groups - NBUF + b, b)

    return gather_kernel, nw, n_chunks


def kernel(indices, embeddings):
    batch, hist = indices.shape
    vocab = embeddings.shape[0]
    n_total = batch * hist

    transpose, tail = _build_transpose(vocab)
    run, nw, n_chunks = _build_gather(n_total)

    main = vocab - tail
    t_rm = transpose(embeddings.T,
                     embeddings[main:].reshape(tail * EMBED_DIM // OUT_W,
                                               OUT_W))
    t_lin = t_rm.reshape(vocab, EMBED_DIM)

    idx3 = indices.reshape(nw, n_chunks, CHUNK)
    out2 = run(idx3, t_lin)
    return out2[:, :EMBED_DIM].reshape(batch, hist, EMBED_DIM)


# speedup vs baseline: 1.1071x; 1.0757x over previous
import functools

import jax
import jax.numpy as jnp
from jax import lax
from jax.experimental import pallas as pl
from jax.experimental.pallas import tpu as pltpu
from jax.experimental.pallas import tpu_sc as plsc

EMBED_DIM = 64
OUT_W = 2 * EMBED_DIM
CHUNK = 128
K = 4
NBUF = 2
TLANES = 384
NLANE = 16


@functools.lru_cache(maxsize=None)
def _build_transpose(vocab):
    info = plsc.get_sparse_core_info()
    nc, ns = info.num_cores, info.num_subcores
    nw = nc * ns
    main = (vocab // CHUNK) * CHUNK
    tail = vocab - main
    n_chunks = main // TLANES
    assert main % TLANES == 0
    n_rounds = -(-n_chunks // nw)

    mesh = plsc.VectorSubcoreMesh(core_axis_name="c", subcore_axis_name="s")

    @functools.partial(
        pl.kernel,
        out_type=jax.ShapeDtypeStruct((vocab // 2, OUT_W), jnp.float32),
        mesh=mesh,
        scratch_types=[
            [pltpu.VMEM((EMBED_DIM, TLANES), jnp.float32) for _ in range(2)],
            [pltpu.VMEM((TLANES // 2, OUT_W), jnp.float32) for _ in range(2)],
            pltpu.VMEM((tail * EMBED_DIM // OUT_W, OUT_W), jnp.float32)
            if tail else None,
            [pltpu.SemaphoreType.DMA for _ in range(2)],
            [pltpu.SemaphoreType.DMA for _ in range(2)],
        ],
        compiler_params=pltpu.CompilerParams(needs_layout_passes=False),
    )
    def transpose_kernel(embT_hbm, tail_hbm, out_hbm, in_v, out_v, tail_v,
                         isem, osem):
        wid = lax.axis_index("s") * nc + lax.axis_index("c")

        ramp = lax.iota(jnp.int32, NLANE)
        row_pat = lax.shift_right_logical(ramp, 1)
        par_pat = lax.mul(lax.rem(ramp, 2), jnp.int32(EMBED_DIM))

        def chunk_of(t):
            return wid + t * nw

        def fire_in(c, b):
            pltpu.async_copy(embT_hbm.at[:, pl.ds(c * TLANES, TLANES)],
                             in_v[b], isem[b])

        def wait_in(c, b):
            pltpu.make_async_copy(embT_hbm.at[:, pl.ds(c * TLANES, TLANES)],
                                  in_v[b], isem[b]).wait()

        def start_out(c, b):
            pltpu.async_copy(out_v[b],
                             out_hbm.at[pl.ds(c * (TLANES // 2), TLANES // 2)],
                             osem[b])

        def wait_out(c, b):
            pltpu.make_async_copy(out_v[b],
                                  out_hbm.at[pl.ds(c * (TLANES // 2),
                                                   TLANES // 2)],
                                  osem[b]).wait()

        def compute(b):
            @pl.loop(0, EMBED_DIM, step=4)
            def _(j):
                for dj in range(4):
                    f_vec = lax.rem(ramp + (j + dj), jnp.int32(EMBED_DIM))
                    cols = par_pat + f_vec
                    for v0 in range(0, TLANES, NLANE):
                        vals = plsc.load_gather(in_v[b], [f_vec, ramp + v0])
                        plsc.store_scatter(
                            out_v[b], [row_pat + (v0 // 2), cols], vals)

        @pl.when(chunk_of(0) < n_chunks)
        def _():
            fire_in(chunk_of(0), 0)

        @pl.loop(0, n_rounds)
        def _(t):
            b = lax.rem(t, 2)
            c = chunk_of(t)

            @pl.when(c < n_chunks)
            def _():
                for bb in range(2):
                    @pl.when(b == bb)
                    def _():
                        wait_in(c, bb)

                        @pl.when(t >= 2)
                        def _():
                            wait_out(chunk_of(t - 2), bb)

                        @pl.when(chunk_of(t + 1) < n_chunks)
                        def _():
                            fire_in(chunk_of(t + 1), 1 - bb)

                        compute(bb)
                        start_out(c, bb)

        my_n = lax.div(n_chunks - wid + nw - 1, nw)

        @pl.loop(0, 2)
        def _(i):
            t_last = my_n - 2 + i

            @pl.when(t_last >= 0)
            def _():
                for bb in range(2):
                    @pl.when(lax.rem(t_last, 2) == bb)
                    def _():
                        wait_out(chunk_of(t_last), bb)

        if tail:
            @pl.when(wid == 0)
            def _():
                pltpu.sync_copy(tail_hbm, tail_v)
                pltpu.sync_copy(
                    tail_v,
                    out_hbm.at[pl.ds(main // 2, tail * EMBED_DIM // OUT_W)])

    return transpose_kernel, tail


@functools.lru_cache(maxsize=None)
def _build_gather(n_total):
    info = plsc.get_sparse_core_info()
    nc, ns = info.num_cores, info.num_subcores
    nw = nc * ns
    per_w = n_total // nw
    group = K * CHUNK
    assert per_w * nw == n_total and per_w % group == 0
    n_chunks = per_w // CHUNK
    n_groups = per_w // group
    assert n_groups % NBUF == 0

    mesh = plsc.VectorSubcoreMesh(core_axis_name="c", subcore_axis_name="s")

    @functools.partial(
        pl.kernel,
        out_type=jax.ShapeDtypeStruct((n_total, OUT_W), jnp.float32),
        mesh=mesh,
        scratch_types=[
            pltpu.VMEM((n_chunks, CHUNK), jnp.int32),
            [pltpu.VMEM((group, EMBED_DIM), jnp.float32) for _ in range(NBUF)],
            [pltpu.SemaphoreType.DMA for _ in range(NBUF)],
            [pltpu.SemaphoreType.DMA for _ in range(NBUF)],
        ],
        compiler_params=pltpu.CompilerParams(use_tc_tiling_on_sc=False),
    )
    def gather_kernel(idx_hbm, table_hbm, out_hbm, idx_v, rows, gsem, wsem):
        wid = lax.axis_index("s") * nc + lax.axis_index("c")
        base = wid * per_w

        pltpu.sync_copy(idx_hbm.at[wid], idx_v)

        def out_slice(g):
            return out_hbm.at[pl.ds(base + g * group, group),
                              pl.ds(0, EMBED_DIM)]

        def fire(g, b):
            for t in range(K):
                pltpu.async_copy(
                    table_hbm.at[idx_v.at[g * K + t]],
                    rows[b].at[pl.ds(t * CHUNK, CHUNK)],
                    gsem[b])

        def drain_gathers(g, b):
            for t in range(K):
                pltpu.make_async_copy(
                    table_hbm.at[idx_v.at[g * K + t]],
                    rows[b].at[pl.ds(t * CHUNK, CHUNK)],
                    gsem[b]).wait()

        def start_write(g, b):
            pltpu.async_copy(rows[b], out_slice(g), wsem[b])

        def wait_write(g, b):
            pltpu.make_async_copy(rows[b], out_slice(g), wsem[b]).wait()

        fire(0, 0)

        @pl.loop(0, n_groups, step=NBUF)
        def _(g0):
            for b in range(NBUF):
                g = g0 + b
                drain_gathers(g, b)
                nb = (b + 1) % NBUF

                @pl.when(g + 1 < n_groups)
                def _():
                    @pl.when(g + 1 >= NBUF)
                    def _():
                        wait_write(g + 1 - NBUF, nb)
                    fire(g + 1, nb)

                start_write(g, b)

        for b in range(NBUF):
            wait_write(n_groups - NBUF + b, b)

    return gather_kernel, nw, n_chunks


def kernel(indices, embeddings):
    batch, hist = indices.shape
    vocab = embeddings.shape[0]
    n_total = batch * hist

    transpose, tail = _build_transpose(vocab)
    run, nw, n_chunks = _build_gather(n_total)

    main = vocab - tail
    t_rm = transpose(embeddings.T,
                     embeddings[main:].reshape(tail * EMBED_DIM // OUT_W,
                                               OUT_W))
    t_lin = t_rm.reshape(vocab, EMBED_DIM)

    idx3 = indices.reshape(nw, n_chunks, CHUNK)
    out2 = run(idx3, t_lin)
    return out2[:, :EMBED_DIM].reshape(batch, hist, EMBED_DIM)
